# Initial kernel scaffold; baseline (speedup 1.0000x reference)
import jax
import jax.numpy as jnp
from jax.experimental import pallas as pl


def kernel(x, W_gate, b_gate, W_noise, b_noise, rng_key):
    raise NotImplementedError("write your pallas kernel here")


# speedup vs baseline: 2.6015x; 2.6015x over previous
import functools

import jax
import jax.numpy as jnp
from jax.experimental import pallas as pl
from jax.experimental.pallas import tpu as pltpu

N_TOKENS = 8192
N_EMBD = 4096
NUM_EXPERTS = 64
TOP_K = 8

BM = 512
BK = 2048
KB = N_EMBD // BK


def _router_block(z):
    bm = z.shape[0]
    cols = jax.lax.broadcasted_iota(jnp.int32, (bm, NUM_EXPERTS), 1)
    work = z
    idxs = []
    mask = jnp.zeros((bm, NUM_EXPERTS), dtype=jnp.bool_)
    for _ in range(TOP_K):
        m = jnp.max(work, axis=1, keepdims=True)
        idx = jnp.min(jnp.where(work == m, cols, NUM_EXPERTS), axis=1,
                      keepdims=True)
        sel = cols == idx
        mask = jnp.logical_or(mask, sel)
        work = jnp.where(sel, -jnp.inf, work)
        idxs.append(idx)
    mx = jnp.max(z, axis=1, keepdims=True)
    e = jnp.where(mask, jnp.exp(z - mx), 0.0)
    router = e / jnp.sum(e, axis=1, keepdims=True)
    idx_out = jnp.concatenate(idxs, axis=1)
    return router, idx_out


def _kernel_body(x_ref, w_ref, b_ref, noise_ref, router_ref, idx_ref, acc_ref):
    k = pl.program_id(1)

    @pl.when(k == 0)
    def _():
        acc_ref[...] = jnp.zeros_like(acc_ref)

    acc_ref[...] += jnp.dot(x_ref[...], w_ref[...],
                            preferred_element_type=jnp.float32)

    @pl.when(k == KB - 1)
    def _():
        acc = acc_ref[...] + b_ref[...]
        logits = acc[:, :NUM_EXPERTS]
        nlog = acc[:, NUM_EXPERTS:]
        softplus = jnp.maximum(nlog, 0.0) + jnp.log1p(jnp.exp(-jnp.abs(nlog)))
        z = logits + noise_ref[...] * softplus
        router, idx_out = _router_block(z)
        router_ref[...] = router
        idx_ref[...] = idx_out


_INTERPRET = False


def kernel(x, W_gate, b_gate, W_noise, b_noise, rng_key):
    w = jnp.concatenate([W_gate, W_noise], axis=1)
    b = jnp.concatenate([b_gate, b_noise])[None, :]
    noise = jax.random.normal(rng_key, (N_TOKENS, NUM_EXPERTS),
                              dtype=jnp.float32)

    grid = (N_TOKENS // BM, KB)
    router, idx = pl.pallas_call(
        _kernel_body,
        grid=grid,
        in_specs=[
            pl.BlockSpec((BM, BK), lambda i, j: (i, j)),
            pl.BlockSpec((BK, 2 * NUM_EXPERTS), lambda i, j: (j, 0)),
            pl.BlockSpec((1, 2 * NUM_EXPERTS), lambda i, j: (0, 0)),
            pl.BlockSpec((BM, NUM_EXPERTS), lambda i, j: (i, 0)),
        ],
        out_specs=[
            pl.BlockSpec((BM, NUM_EXPERTS), lambda i, j: (i, 0)),
            pl.BlockSpec((BM, TOP_K), lambda i, j: (i, 0)),
        ],
        out_shape=[
            jax.ShapeDtypeStruct((N_TOKENS, NUM_EXPERTS), jnp.float32),
            jax.ShapeDtypeStruct((N_TOKENS, TOP_K), jnp.int32),
        ],
        scratch_shapes=[pltpu.VMEM((BM, 2 * NUM_EXPERTS), jnp.float32)],
        compiler_params=pltpu.CompilerParams(
            dimension_semantics=("parallel", "arbitrary")),
        interpret=_INTERPRET,
    )(x, w, b, noise)
    return (router, idx)
